# Initial kernel scaffold; baseline (speedup 1.0000x reference)
import jax
import jax.numpy as jnp
from jax.experimental import pallas as pl


def kernel(adj_m_c1, adj_m_c2, adj_a_c1, adj_a_c2, adj_m_t1, adj_m_t2, adj_a_t1, adj_a_t2, mashup_call_W, api_call_W, mashup_tag_W, api_tag_W, u_weights, i_weights, m_t_weights, a_t_weights, mashup_view_weights, api_view_weights, m_fc1_w, m_fc1_b, m_ln_g, m_ln_b, m_fc2_w, m_fc2_b, a_fc1_w, a_fc1_b, a_ln_g, a_ln_b, a_fc2_w, a_fc2_b, m_pred_w, m_pred_b, a_pred_w, a_pred_b):
    raise NotImplementedError("write your pallas kernel here")


# speedup vs baseline: 1.4057x; 1.4057x over previous
import functools

import jax
import jax.numpy as jnp
from jax.experimental import pallas as pl
from jax.experimental.pallas import tpu as pltpu

F32 = jnp.float32
BF16 = jnp.bfloat16
_BM = 256


def _stage1_body(a_ref, b_ref, o_ref):
    a = a_ref[...].astype(BF16)
    o_ref[...] = jnp.dot(a, b_ref[...], preferred_element_type=F32).astype(BF16)


def _mm_bf16(a, b_bf16):
    m, k = a.shape
    _, n = b_bf16.shape
    return pl.pallas_call(
        _stage1_body,
        grid=(m // _BM,),
        in_specs=[
            pl.BlockSpec((_BM, k), lambda i: (i, 0)),
            pl.BlockSpec((k, n), lambda i: (0, 0)),
        ],
        out_specs=pl.BlockSpec((_BM, n), lambda i: (i, 0)),
        out_shape=jax.ShapeDtypeStruct((m, n), BF16),
        compiler_params=pltpu.CompilerParams(
            dimension_semantics=("parallel",)),
    )(a, b_bf16)


def _stage2_body(w_ref, a_ref, t_ref, res_ref, o_ref):
    a = a_ref[...].astype(BF16)
    p = jnp.dot(a, t_ref[...], preferred_element_type=F32)
    o_ref[...] = w_ref[0] * res_ref[...] + w_ref[1] * p


def _propagate_combine(w2, adj1, t_bf16, init_w):
    m, k = adj1.shape
    _, n = t_bf16.shape
    return pl.pallas_call(
        _stage2_body,
        grid=(m // _BM,),
        in_specs=[
            pl.BlockSpec(memory_space=pltpu.SMEM),
            pl.BlockSpec((_BM, k), lambda i: (i, 0)),
            pl.BlockSpec((k, n), lambda i: (0, 0)),
            pl.BlockSpec((_BM, n), lambda i: (i, 0)),
        ],
        out_specs=pl.BlockSpec((_BM, n), lambda i: (i, 0)),
        out_shape=jax.ShapeDtypeStruct((m, n), F32),
        compiler_params=pltpu.CompilerParams(
            dimension_semantics=("parallel",)),
    )(w2, adj1, t_bf16, init_w)


def _emblog_body(vw_ref, ea_ref, eb_ref, pw_ref, pb_ref, emb_ref, log_ref):
    e = vw_ref[0] * ea_ref[...] + vw_ref[1] * eb_ref[...]
    emb_ref[...] = e
    log_ref[...] = (
        jnp.dot(e.astype(BF16), pw_ref[...], preferred_element_type=F32)
        + pb_ref[...])


def _emb_and_logits(vw, emb_a, emb_b, pred_w_bf16, pred_b):
    m, d = emb_a.shape
    _, n = pred_w_bf16.shape
    return pl.pallas_call(
        _emblog_body,
        grid=(m // _BM,),
        in_specs=[
            pl.BlockSpec(memory_space=pltpu.SMEM),
            pl.BlockSpec((_BM, d), lambda i: (i, 0)),
            pl.BlockSpec((_BM, d), lambda i: (i, 0)),
            pl.BlockSpec((d, n), lambda i: (0, 0)),
            pl.BlockSpec((1, n), lambda i: (0, 0)),
        ],
        out_specs=[
            pl.BlockSpec((_BM, d), lambda i: (i, 0)),
            pl.BlockSpec((_BM, n), lambda i: (i, 0)),
        ],
        out_shape=[
            jax.ShapeDtypeStruct((m, d), F32),
            jax.ShapeDtypeStruct((m, n), F32),
        ],
        compiler_params=pltpu.CompilerParams(
            dimension_semantics=("parallel",)),
    )(vw, emb_a, emb_b, pred_w_bf16, pred_b)


def _mlp_body(x_ref, w1_ref, b1_ref, g_ref, be_ref, w2_ref, b2_ref, o_ref):
    x = x_ref[...].astype(BF16)
    h = jnp.dot(x, w1_ref[...], preferred_element_type=F32) + b1_ref[...]
    mu = jnp.mean(h, axis=-1, keepdims=True)
    var = jnp.mean((h - mu) ** 2, axis=-1, keepdims=True)
    h = (h - mu) * jax.lax.rsqrt(var + 1e-5) * g_ref[...] + be_ref[...]
    h = 0.5 * h * (1.0 + jax.lax.erf(h * 0.7071067811865476))
    o_ref[...] = (
        jnp.dot(h.astype(BF16), w2_ref[...], preferred_element_type=F32)
        + b2_ref[...])


def _proj_mlp(x, w1_bf16, b1, g, be, w2_bf16, b2):
    m, d = x.shape
    return pl.pallas_call(
        _mlp_body,
        grid=(m // _BM,),
        in_specs=[
            pl.BlockSpec((_BM, d), lambda i: (i, 0)),
            pl.BlockSpec((d, d), lambda i: (0, 0)),
            pl.BlockSpec((1, d), lambda i: (0, 0)),
            pl.BlockSpec((1, d), lambda i: (0, 0)),
            pl.BlockSpec((1, d), lambda i: (0, 0)),
            pl.BlockSpec((d, d), lambda i: (0, 0)),
            pl.BlockSpec((1, d), lambda i: (0, 0)),
        ],
        out_specs=pl.BlockSpec((_BM, d), lambda i: (i, 0)),
        out_shape=jax.ShapeDtypeStruct((m, d), F32),
        compiler_params=pltpu.CompilerParams(
            dimension_semantics=("parallel",)),
    )(x, w1_bf16, b1, g, be, w2_bf16, b2)


def kernel(adj_m_c1, adj_m_c2, adj_a_c1, adj_a_c2, adj_m_t1, adj_m_t2,
           adj_a_t1, adj_a_t2, mashup_call_W, api_call_W, mashup_tag_W,
           api_tag_W, u_weights, i_weights, m_t_weights, a_t_weights,
           mashup_view_weights, api_view_weights, m_fc1_w, m_fc1_b,
           m_ln_g, m_ln_b, m_fc2_w, m_fc2_b, a_fc1_w, a_fc1_b, a_ln_g,
           a_ln_b, a_fc2_w, a_fc2_b, m_pred_w, m_pred_b, a_pred_w,
           a_pred_b):
    uw = jax.nn.softmax(u_weights, axis=0)
    iw = jax.nn.softmax(i_weights, axis=0)
    mtw = jax.nn.softmax(m_t_weights, axis=0)
    atw = jax.nn.softmax(a_t_weights, axis=0)
    mvw = jax.nn.softmax(mashup_view_weights, axis=0)
    avw = jax.nn.softmax(api_view_weights, axis=0)

    t_mc = _mm_bf16(adj_m_c2, mashup_call_W.astype(BF16))
    t_ac = _mm_bf16(adj_a_c2, api_call_W.astype(BF16))
    t_mt = _mm_bf16(adj_m_t2, mashup_tag_W.astype(BF16))
    t_at = _mm_bf16(adj_a_t2, api_tag_W.astype(BF16))

    emb_mc = _propagate_combine(uw, adj_m_c1, t_mc, mashup_call_W)
    emb_ac = _propagate_combine(iw, adj_a_c1, t_ac, api_call_W)
    emb_mt = _propagate_combine(mtw, adj_m_t1, t_mt, mashup_tag_W)
    emb_at = _propagate_combine(atw, adj_a_t1, t_at, api_tag_W)

    mashup_emb, m_logits = _emb_and_logits(
        mvw, emb_mc, emb_mt, m_pred_w.astype(BF16), m_pred_b.reshape(1, -1))
    api_emb, a_logits = _emb_and_logits(
        avw, emb_ac, emb_at, a_pred_w.astype(BF16), a_pred_b.reshape(1, -1))

    m_w1 = m_fc1_w.astype(BF16)
    m_w2 = m_fc2_w.astype(BF16)
    a_w1 = a_fc1_w.astype(BF16)
    a_w2 = a_fc2_w.astype(BF16)
    m_b1 = m_fc1_b.reshape(1, -1)
    m_b2 = m_fc2_b.reshape(1, -1)
    a_b1 = a_fc1_b.reshape(1, -1)
    a_b2 = a_fc2_b.reshape(1, -1)
    m_g = m_ln_g.reshape(1, -1)
    m_be = m_ln_b.reshape(1, -1)
    a_g = a_ln_g.reshape(1, -1)
    a_be = a_ln_b.reshape(1, -1)

    mashup_call_proj = _proj_mlp(emb_mc, m_w1, m_b1, m_g, m_be, m_w2, m_b2)
    mashup_tag_proj = _proj_mlp(emb_mt, m_w1, m_b1, m_g, m_be, m_w2, m_b2)
    api_call_proj = _proj_mlp(emb_ac, a_w1, a_b1, a_g, a_be, a_w2, a_b2)
    api_tag_proj = _proj_mlp(emb_at, a_w1, a_b1, a_g, a_be, a_w2, a_b2)

    return (mashup_emb, api_emb, mashup_call_proj, mashup_tag_proj,
            api_call_proj, api_tag_proj, m_logits, a_logits)


# speedup vs baseline: 1.8409x; 1.3096x over previous
import functools

import jax
import jax.numpy as jnp
from jax.experimental import pallas as pl
from jax.experimental.pallas import tpu as pltpu

F32 = jnp.float32
BF16 = jnp.bfloat16
_BM = 256


def _stage1_body(a_ref, b_ref, o_ref):
    a = a_ref[...].astype(BF16)
    o_ref[...] = jnp.dot(a, b_ref[...], preferred_element_type=F32).astype(BF16)


def _mm_bf16(a, b_bf16):
    m, k = a.shape
    _, n = b_bf16.shape
    return pl.pallas_call(
        _stage1_body,
        grid=(m // _BM,),
        in_specs=[
            pl.BlockSpec((_BM, k), lambda i: (i, 0)),
            pl.BlockSpec((k, n), lambda i: (0, 0)),
        ],
        out_specs=pl.BlockSpec((_BM, n), lambda i: (i, 0)),
        out_shape=jax.ShapeDtypeStruct((m, n), BF16),
        compiler_params=pltpu.CompilerParams(
            dimension_semantics=("parallel",)),
    )(a, b_bf16)


def _mlp_tile(x, w1_ref, b1_ref, g_ref, be_ref, w2_ref, b2_ref):
    h = (jnp.dot(x.astype(BF16), w1_ref[...], preferred_element_type=F32)
         + b1_ref[...])
    mu = jnp.mean(h, axis=-1, keepdims=True)
    var = jnp.mean((h - mu) ** 2, axis=-1, keepdims=True)
    h = (h - mu) * jax.lax.rsqrt(var + 1e-5) * g_ref[...] + be_ref[...]
    h = 0.5 * h * (1.0 + jax.lax.erf(h * 0.7071067811865476))
    return (jnp.dot(h.astype(BF16), w2_ref[...], preferred_element_type=F32)
            + b2_ref[...])


def _entity_body(sc_ref, adj_c_ref, tc_ref, wc_ref, adj_t_ref, tt_ref, wt_ref,
                 w1_ref, b1_ref, g_ref, be_ref, w2_ref, b2_ref,
                 pw_ref, pb_ref,
                 emb_ref, log_ref, pc_ref, pt_ref):
    emb_c = sc_ref[0] * wc_ref[...] + sc_ref[1] * jnp.dot(
        adj_c_ref[...].astype(BF16), tc_ref[...], preferred_element_type=F32)
    emb_t = sc_ref[2] * wt_ref[...] + sc_ref[3] * jnp.dot(
        adj_t_ref[...].astype(BF16), tt_ref[...], preferred_element_type=F32)
    e = sc_ref[4] * emb_c + sc_ref[5] * emb_t
    emb_ref[...] = e
    log_ref[...] = (
        jnp.dot(e.astype(BF16), pw_ref[...], preferred_element_type=F32)
        + pb_ref[...])
    pc_ref[...] = _mlp_tile(emb_c, w1_ref, b1_ref, g_ref, be_ref, w2_ref,
                            b2_ref)
    pt_ref[...] = _mlp_tile(emb_t, w1_ref, b1_ref, g_ref, be_ref, w2_ref,
                            b2_ref)


def _entity_block(scalars, adj_c, t_c, w_c, adj_t, t_t, w_t,
                  fc1_w, fc1_b, ln_g, ln_b, fc2_w, fc2_b, pred_w, pred_b):
    m, kc = adj_c.shape
    _, kt = adj_t.shape
    d = w_c.shape[1]
    t = pred_w.shape[1]
    full = lambda shape: pl.BlockSpec(shape, lambda i: (0, 0))
    row = lambda shape: pl.BlockSpec(shape, lambda i: (i, 0))
    return pl.pallas_call(
        _entity_body,
        grid=(m // _BM,),
        in_specs=[
            pl.BlockSpec(memory_space=pltpu.SMEM),
            row((_BM, kc)), full((kc, d)), row((_BM, d)),
            row((_BM, kt)), full((kt, d)), row((_BM, d)),
            full((d, d)), full((1, d)), full((1, d)), full((1, d)),
            full((d, d)), full((1, d)),
            full((d, t)), full((1, t)),
        ],
        out_specs=[
            row((_BM, d)), row((_BM, t)), row((_BM, d)), row((_BM, d)),
        ],
        out_shape=[
            jax.ShapeDtypeStruct((m, d), F32),
            jax.ShapeDtypeStruct((m, t), F32),
            jax.ShapeDtypeStruct((m, d), F32),
            jax.ShapeDtypeStruct((m, d), F32),
        ],
        compiler_params=pltpu.CompilerParams(
            dimension_semantics=("parallel",)),
    )(scalars, adj_c, t_c, w_c, adj_t, t_t, w_t,
      fc1_w, fc1_b, ln_g, ln_b, fc2_w, fc2_b, pred_w, pred_b)


def kernel(adj_m_c1, adj_m_c2, adj_a_c1, adj_a_c2, adj_m_t1, adj_m_t2,
           adj_a_t1, adj_a_t2, mashup_call_W, api_call_W, mashup_tag_W,
           api_tag_W, u_weights, i_weights, m_t_weights, a_t_weights,
           mashup_view_weights, api_view_weights, m_fc1_w, m_fc1_b,
           m_ln_g, m_ln_b, m_fc2_w, m_fc2_b, a_fc1_w, a_fc1_b, a_ln_g,
           a_ln_b, a_fc2_w, a_fc2_b, m_pred_w, m_pred_b, a_pred_w,
           a_pred_b):
    uw = jax.nn.softmax(u_weights, axis=0)
    iw = jax.nn.softmax(i_weights, axis=0)
    mtw = jax.nn.softmax(m_t_weights, axis=0)
    atw = jax.nn.softmax(a_t_weights, axis=0)
    mvw = jax.nn.softmax(mashup_view_weights, axis=0)
    avw = jax.nn.softmax(api_view_weights, axis=0)

    t_mc = _mm_bf16(adj_m_c2, mashup_call_W.astype(BF16))
    t_ac = _mm_bf16(adj_a_c2, api_call_W.astype(BF16))
    t_mt = _mm_bf16(adj_m_t2, mashup_tag_W.astype(BF16))
    t_at = _mm_bf16(adj_a_t2, api_tag_W.astype(BF16))

    m_scal = jnp.concatenate([uw, mtw, mvw])
    a_scal = jnp.concatenate([iw, atw, avw])

    mashup_emb, m_logits, mashup_call_proj, mashup_tag_proj = _entity_block(
        m_scal, adj_m_c1, t_mc, mashup_call_W, adj_m_t1, t_mt, mashup_tag_W,
        m_fc1_w.astype(BF16), m_fc1_b.reshape(1, -1),
        m_ln_g.reshape(1, -1), m_ln_b.reshape(1, -1),
        m_fc2_w.astype(BF16), m_fc2_b.reshape(1, -1),
        m_pred_w.astype(BF16), m_pred_b.reshape(1, -1))
    api_emb, a_logits, api_call_proj, api_tag_proj = _entity_block(
        a_scal, adj_a_c1, t_ac, api_call_W, adj_a_t1, t_at, api_tag_W,
        a_fc1_w.astype(BF16), a_fc1_b.reshape(1, -1),
        a_ln_g.reshape(1, -1), a_ln_b.reshape(1, -1),
        a_fc2_w.astype(BF16), a_fc2_b.reshape(1, -1),
        a_pred_w.astype(BF16), a_pred_b.reshape(1, -1))

    return (mashup_emb, api_emb, mashup_call_proj, mashup_tag_proj,
            api_call_proj, api_tag_proj, m_logits, a_logits)


# speedup vs baseline: 1.9040x; 1.0343x over previous
import functools

import jax
import jax.numpy as jnp
from jax.experimental import pallas as pl
from jax.experimental.pallas import tpu as pltpu

F32 = jnp.float32
BF16 = jnp.bfloat16
_BM = 512


def _stage1_body(a_ref, b_ref, o_ref):
    a = a_ref[...].astype(BF16)
    o_ref[...] = jnp.dot(a, b_ref[...], preferred_element_type=F32).astype(BF16)


def _mm_bf16(a, b_bf16):
    m, k = a.shape
    _, n = b_bf16.shape
    return pl.pallas_call(
        _stage1_body,
        grid=(m // _BM,),
        in_specs=[
            pl.BlockSpec((_BM, k), lambda i: (i, 0)),
            pl.BlockSpec((k, n), lambda i: (0, 0)),
        ],
        out_specs=pl.BlockSpec((_BM, n), lambda i: (i, 0)),
        out_shape=jax.ShapeDtypeStruct((m, n), BF16),
        compiler_params=pltpu.CompilerParams(
            dimension_semantics=("parallel",)),
    )(a, b_bf16)


def _mlp_tile(x, w1_ref, b1_ref, g_ref, be_ref, w2_ref, b2_ref):
    h = (jnp.dot(x.astype(BF16), w1_ref[...], preferred_element_type=F32)
         + b1_ref[...])
    mu = jnp.mean(h, axis=-1, keepdims=True)
    var = jnp.mean((h - mu) ** 2, axis=-1, keepdims=True)
    h = (h - mu) * jax.lax.rsqrt(var + 1e-5) * g_ref[...] + be_ref[...]
    h = 0.5 * h * (1.0 + jax.lax.erf(h * 0.7071067811865476))
    return (jnp.dot(h.astype(BF16), w2_ref[...], preferred_element_type=F32)
            + b2_ref[...])


def _entity_body(sc_ref, adj_c_ref, tc_ref, wc_ref, adj_t_ref, tt_ref, wt_ref,
                 w1_ref, b1_ref, g_ref, be_ref, w2_ref, b2_ref,
                 pw_ref, pb_ref,
                 emb_ref, log_ref, pc_ref, pt_ref):
    emb_c = sc_ref[0] * wc_ref[...] + sc_ref[1] * jnp.dot(
        adj_c_ref[...].astype(BF16), tc_ref[...], preferred_element_type=F32)
    emb_t = sc_ref[2] * wt_ref[...] + sc_ref[3] * jnp.dot(
        adj_t_ref[...].astype(BF16), tt_ref[...], preferred_element_type=F32)
    e = sc_ref[4] * emb_c + sc_ref[5] * emb_t
    emb_ref[...] = e
    log_ref[...] = (
        jnp.dot(e.astype(BF16), pw_ref[...], preferred_element_type=F32)
        + pb_ref[...])
    pc_ref[...] = _mlp_tile(emb_c, w1_ref, b1_ref, g_ref, be_ref, w2_ref,
                            b2_ref)
    pt_ref[...] = _mlp_tile(emb_t, w1_ref, b1_ref, g_ref, be_ref, w2_ref,
                            b2_ref)


def _entity_block(scalars, adj_c, t_c, w_c, adj_t, t_t, w_t,
                  fc1_w, fc1_b, ln_g, ln_b, fc2_w, fc2_b, pred_w, pred_b):
    m, kc = adj_c.shape
    _, kt = adj_t.shape
    d = w_c.shape[1]
    t = pred_w.shape[1]
    full = lambda shape: pl.BlockSpec(shape, lambda i: (0, 0))
    row = lambda shape: pl.BlockSpec(shape, lambda i: (i, 0))
    return pl.pallas_call(
        _entity_body,
        grid=(m // _BM,),
        in_specs=[
            pl.BlockSpec(memory_space=pltpu.SMEM),
            row((_BM, kc)), full((kc, d)), row((_BM, d)),
            row((_BM, kt)), full((kt, d)), row((_BM, d)),
            full((d, d)), full((1, d)), full((1, d)), full((1, d)),
            full((d, d)), full((1, d)),
            full((d, t)), full((1, t)),
        ],
        out_specs=[
            row((_BM, d)), row((_BM, t)), row((_BM, d)), row((_BM, d)),
        ],
        out_shape=[
            jax.ShapeDtypeStruct((m, d), F32),
            jax.ShapeDtypeStruct((m, t), F32),
            jax.ShapeDtypeStruct((m, d), F32),
            jax.ShapeDtypeStruct((m, d), F32),
        ],
        compiler_params=pltpu.CompilerParams(
            dimension_semantics=("parallel",)),
    )(scalars, adj_c, t_c, w_c, adj_t, t_t, w_t,
      fc1_w, fc1_b, ln_g, ln_b, fc2_w, fc2_b, pred_w, pred_b)


def kernel(adj_m_c1, adj_m_c2, adj_a_c1, adj_a_c2, adj_m_t1, adj_m_t2,
           adj_a_t1, adj_a_t2, mashup_call_W, api_call_W, mashup_tag_W,
           api_tag_W, u_weights, i_weights, m_t_weights, a_t_weights,
           mashup_view_weights, api_view_weights, m_fc1_w, m_fc1_b,
           m_ln_g, m_ln_b, m_fc2_w, m_fc2_b, a_fc1_w, a_fc1_b, a_ln_g,
           a_ln_b, a_fc2_w, a_fc2_b, m_pred_w, m_pred_b, a_pred_w,
           a_pred_b):
    uw = jax.nn.softmax(u_weights, axis=0)
    iw = jax.nn.softmax(i_weights, axis=0)
    mtw = jax.nn.softmax(m_t_weights, axis=0)
    atw = jax.nn.softmax(a_t_weights, axis=0)
    mvw = jax.nn.softmax(mashup_view_weights, axis=0)
    avw = jax.nn.softmax(api_view_weights, axis=0)

    t_mc = _mm_bf16(adj_m_c2, mashup_call_W.astype(BF16))
    t_ac = _mm_bf16(adj_a_c2, api_call_W.astype(BF16))
    t_mt = _mm_bf16(adj_m_t2, mashup_tag_W.astype(BF16))
    t_at = _mm_bf16(adj_a_t2, api_tag_W.astype(BF16))

    m_scal = jnp.concatenate([uw, mtw, mvw])
    a_scal = jnp.concatenate([iw, atw, avw])

    mashup_emb, m_logits, mashup_call_proj, mashup_tag_proj = _entity_block(
        m_scal, adj_m_c1, t_mc, mashup_call_W, adj_m_t1, t_mt, mashup_tag_W,
        m_fc1_w.astype(BF16), m_fc1_b.reshape(1, -1),
        m_ln_g.reshape(1, -1), m_ln_b.reshape(1, -1),
        m_fc2_w.astype(BF16), m_fc2_b.reshape(1, -1),
        m_pred_w.astype(BF16), m_pred_b.reshape(1, -1))
    api_emb, a_logits, api_call_proj, api_tag_proj = _entity_block(
        a_scal, adj_a_c1, t_ac, api_call_W, adj_a_t1, t_at, api_tag_W,
        a_fc1_w.astype(BF16), a_fc1_b.reshape(1, -1),
        a_ln_g.reshape(1, -1), a_ln_b.reshape(1, -1),
        a_fc2_w.astype(BF16), a_fc2_b.reshape(1, -1),
        a_pred_w.astype(BF16), a_pred_b.reshape(1, -1))

    return (mashup_emb, api_emb, mashup_call_proj, mashup_tag_proj,
            api_call_proj, api_tag_proj, m_logits, a_logits)


# speedup vs baseline: 2.1092x; 1.1077x over previous
import functools

import jax
import jax.numpy as jnp
from jax.experimental import pallas as pl
from jax.experimental.pallas import tpu as pltpu

F32 = jnp.float32
BF16 = jnp.bfloat16
_BM = 512


def _stage1_body(a_ref, b_ref, o_ref, bscr_ref):
    @pl.when(pl.program_id(0) == 0)
    def _():
        bscr_ref[...] = b_ref[...].astype(BF16)

    a = a_ref[...].astype(BF16)
    o_ref[...] = jnp.dot(a, bscr_ref[...],
                         preferred_element_type=F32).astype(BF16)


def _mm_bf16(a, b):
    m, k = a.shape
    _, n = b.shape
    return pl.pallas_call(
        _stage1_body,
        grid=(m // _BM,),
        in_specs=[
            pl.BlockSpec((_BM, k), lambda i: (i, 0)),
            pl.BlockSpec((k, n), lambda i: (0, 0)),
        ],
        out_specs=pl.BlockSpec((_BM, n), lambda i: (i, 0)),
        out_shape=jax.ShapeDtypeStruct((m, n), BF16),
        scratch_shapes=[pltpu.VMEM((k, n), BF16)],
        compiler_params=pltpu.CompilerParams(
            dimension_semantics=("arbitrary",)),
    )(a, b)


def _mlp_tile(x, w1_ref, b1_ref, g_ref, be_ref, w2_ref, b2_ref):
    h = (jnp.dot(x.astype(BF16), w1_ref[...], preferred_element_type=F32)
         + b1_ref[...])
    mu = jnp.mean(h, axis=-1, keepdims=True)
    var = jnp.mean((h - mu) ** 2, axis=-1, keepdims=True)
    h = (h - mu) * jax.lax.rsqrt(var + 1e-5) * g_ref[...] + be_ref[...]
    h = 0.5 * h * (1.0 + jax.lax.erf(h * 0.7071067811865476))
    return (jnp.dot(h.astype(BF16), w2_ref[...], preferred_element_type=F32)
            + b2_ref[...])


def _entity_body(sc_ref, adj_c_ref, tc_ref, wc_ref, adj_t_ref, tt_ref, wt_ref,
                 w1_ref, b1_ref, g_ref, be_ref, w2_ref, b2_ref,
                 pw_ref, pb_ref,
                 emb_ref, log_ref, pc_ref, pt_ref,
                 w1s_ref, w2s_ref, pws_ref):
    @pl.when(pl.program_id(0) == 0)
    def _():
        w1s_ref[...] = w1_ref[...].astype(BF16)
        w2s_ref[...] = w2_ref[...].astype(BF16)
        pws_ref[...] = pw_ref[...].astype(BF16)

    emb_c = sc_ref[0] * wc_ref[...] + sc_ref[1] * jnp.dot(
        adj_c_ref[...].astype(BF16), tc_ref[...], preferred_element_type=F32)
    emb_t = sc_ref[2] * wt_ref[...] + sc_ref[3] * jnp.dot(
        adj_t_ref[...].astype(BF16), tt_ref[...], preferred_element_type=F32)
    e = sc_ref[4] * emb_c + sc_ref[5] * emb_t
    emb_ref[...] = e
    log_ref[...] = (
        jnp.dot(e.astype(BF16), pws_ref[...], preferred_element_type=F32)
        + pb_ref[...])
    pc_ref[...] = _mlp_tile(emb_c, w1s_ref, b1_ref, g_ref, be_ref, w2s_ref,
                            b2_ref)
    pt_ref[...] = _mlp_tile(emb_t, w1s_ref, b1_ref, g_ref, be_ref, w2s_ref,
                            b2_ref)


def _entity_block(scalars, adj_c, t_c, w_c, adj_t, t_t, w_t,
                  fc1_w, fc1_b, ln_g, ln_b, fc2_w, fc2_b, pred_w, pred_b):
    m, kc = adj_c.shape
    _, kt = adj_t.shape
    d = w_c.shape[1]
    t = pred_w.shape[1]
    full = lambda shape: pl.BlockSpec(shape, lambda i: (0, 0))
    row = lambda shape: pl.BlockSpec(shape, lambda i: (i, 0))
    return pl.pallas_call(
        _entity_body,
        grid=(m // _BM,),
        in_specs=[
            pl.BlockSpec(memory_space=pltpu.SMEM),
            row((_BM, kc)), full((kc, d)), row((_BM, d)),
            row((_BM, kt)), full((kt, d)), row((_BM, d)),
            full((d, d)), full((1, d)), full((1, d)), full((1, d)),
            full((d, d)), full((1, d)),
            full((d, t)), full((1, t)),
        ],
        out_specs=[
            row((_BM, d)), row((_BM, t)), row((_BM, d)), row((_BM, d)),
        ],
        out_shape=[
            jax.ShapeDtypeStruct((m, d), F32),
            jax.ShapeDtypeStruct((m, t), F32),
            jax.ShapeDtypeStruct((m, d), F32),
            jax.ShapeDtypeStruct((m, d), F32),
        ],
        scratch_shapes=[
            pltpu.VMEM((d, d), BF16),
            pltpu.VMEM((d, d), BF16),
            pltpu.VMEM((d, t), BF16),
        ],
        compiler_params=pltpu.CompilerParams(
            dimension_semantics=("arbitrary",)),
    )(scalars, adj_c, t_c, w_c, adj_t, t_t, w_t,
      fc1_w, fc1_b, ln_g, ln_b, fc2_w, fc2_b, pred_w, pred_b)


def kernel(adj_m_c1, adj_m_c2, adj_a_c1, adj_a_c2, adj_m_t1, adj_m_t2,
           adj_a_t1, adj_a_t2, mashup_call_W, api_call_W, mashup_tag_W,
           api_tag_W, u_weights, i_weights, m_t_weights, a_t_weights,
           mashup_view_weights, api_view_weights, m_fc1_w, m_fc1_b,
           m_ln_g, m_ln_b, m_fc2_w, m_fc2_b, a_fc1_w, a_fc1_b, a_ln_g,
           a_ln_b, a_fc2_w, a_fc2_b, m_pred_w, m_pred_b, a_pred_w,
           a_pred_b):
    uw = jax.nn.softmax(u_weights, axis=0)
    iw = jax.nn.softmax(i_weights, axis=0)
    mtw = jax.nn.softmax(m_t_weights, axis=0)
    atw = jax.nn.softmax(a_t_weights, axis=0)
    mvw = jax.nn.softmax(mashup_view_weights, axis=0)
    avw = jax.nn.softmax(api_view_weights, axis=0)

    t_mc = _mm_bf16(adj_m_c2, mashup_call_W)
    t_ac = _mm_bf16(adj_a_c2, api_call_W)
    t_mt = _mm_bf16(adj_m_t2, mashup_tag_W)
    t_at = _mm_bf16(adj_a_t2, api_tag_W)

    m_scal = jnp.concatenate([uw, mtw, mvw])
    a_scal = jnp.concatenate([iw, atw, avw])

    mashup_emb, m_logits, mashup_call_proj, mashup_tag_proj = _entity_block(
        m_scal, adj_m_c1, t_mc, mashup_call_W, adj_m_t1, t_mt, mashup_tag_W,
        m_fc1_w, m_fc1_b.reshape(1, -1),
        m_ln_g.reshape(1, -1), m_ln_b.reshape(1, -1),
        m_fc2_w, m_fc2_b.reshape(1, -1),
        m_pred_w, m_pred_b.reshape(1, -1))
    api_emb, a_logits, api_call_proj, api_tag_proj = _entity_block(
        a_scal, adj_a_c1, t_ac, api_call_W, adj_a_t1, t_at, api_tag_W,
        a_fc1_w, a_fc1_b.reshape(1, -1),
        a_ln_g.reshape(1, -1), a_ln_b.reshape(1, -1),
        a_fc2_w, a_fc2_b.reshape(1, -1),
        a_pred_w, a_pred_b.reshape(1, -1))

    return (mashup_emb, api_emb, mashup_call_proj, mashup_tag_proj,
            api_call_proj, api_tag_proj, m_logits, a_logits)
